# baseline (speedup 1.0000x reference)
import jax
import jax.numpy as jnp
from jax.experimental import pallas as pl

E = 8
C = 512
D_IN = 2048
D_OUT = 1024
B = 1
S = 2048
T = B * S


def _fused_kernel(x_ref, cw_ref, w_ref, b_ref, out_ref):
    e = pl.program_id(0)
    tmp = jnp.dot(x_ref[0], w_ref[0], preferred_element_type=jnp.float32)
    tmp = (tmp + b_ref[0]).astype(jnp.bfloat16)
    part = jnp.dot(cw_ref[...], tmp, preferred_element_type=jnp.float32)

    @pl.when(e == 0)
    def _init():
        out_ref[...] = part

    @pl.when(e != 0)
    def _acc():
        out_ref[...] += part


def kernel(inputs, combine_weights, weight, bias):
    x = inputs.astype(jnp.bfloat16)
    cw = combine_weights.astype(jnp.bfloat16)
    w = weight.astype(jnp.bfloat16)
    b = bias.reshape(E, 1, D_OUT)

    out = pl.pallas_call(
        _fused_kernel,
        grid=(E,),
        in_specs=[
            pl.BlockSpec((1, C, D_IN), lambda e: (e, 0, 0)),
            pl.BlockSpec((T, C), lambda e: (0, e)),
            pl.BlockSpec((1, D_IN, D_OUT), lambda e: (e, 0, 0)),
            pl.BlockSpec((1, 1, D_OUT), lambda e: (e, 0, 0)),
        ],
        out_specs=pl.BlockSpec((T, D_OUT), lambda e: (0, 0)),
        out_shape=jax.ShapeDtypeStruct((T, D_OUT), jnp.float32),
    )(x, cw, w, b)
    return out.reshape(B, S, D_OUT)


# speedup vs baseline: 2.1171x; 2.1171x over previous
import jax
import jax.numpy as jnp
from jax.experimental import pallas as pl

E = 8
C = 512
D_IN = 2048
D_OUT = 1024
B = 1
S = 2048
T = B * S


def _fused_kernel(x_ref, cw_ref, w_ref, b_ref, out_ref):
    e = pl.program_id(0)
    x = x_ref[0].astype(jnp.bfloat16)
    w = w_ref[0].astype(jnp.bfloat16)
    tmp = jnp.dot(x, w, preferred_element_type=jnp.float32)
    tmp = (tmp + b_ref[0]).astype(jnp.bfloat16)
    cw = cw_ref[...].astype(jnp.bfloat16)
    part = jnp.dot(cw, tmp, preferred_element_type=jnp.float32)

    @pl.when(e == 0)
    def _init():
        out_ref[...] = part

    @pl.when(e != 0)
    def _acc():
        out_ref[...] += part


def kernel(inputs, combine_weights, weight, bias):
    b = bias.reshape(E, 1, D_OUT)

    out = pl.pallas_call(
        _fused_kernel,
        grid=(E,),
        in_specs=[
            pl.BlockSpec((1, C, D_IN), lambda e: (e, 0, 0)),
            pl.BlockSpec((T, C), lambda e: (0, e)),
            pl.BlockSpec((1, D_IN, D_OUT), lambda e: (e, 0, 0)),
            pl.BlockSpec((1, 1, D_OUT), lambda e: (e, 0, 0)),
        ],
        out_specs=pl.BlockSpec((T, D_OUT), lambda e: (0, 0)),
        out_shape=jax.ShapeDtypeStruct((T, D_OUT), jnp.float32),
    )(inputs, combine_weights, weight, b)
    return out.reshape(B, S, D_OUT)
